# Initial kernel scaffold; baseline (speedup 1.0000x reference)
import jax
import jax.numpy as jnp
from jax.experimental import pallas as pl


def kernel(item_embedding_g, user_embedding_g, gi_rows, gi_cols, gi_vals, gu_rows, gu_cols, gu_vals, A_rows, A_cols, A_vals, u_w, i_w):
    raise NotImplementedError("write your pallas kernel here")


# speedup vs baseline: 5.0035x; 5.0035x over previous
import functools

import jax
import jax.numpy as jnp
from jax import lax
from jax.experimental import pallas as pl
from jax.experimental.pallas import tpu as pltpu
from jax.experimental.pallas import tpu_sc as plsc

NB = 2
NG = 2
NU = 10000
NI = 10000
NN = 10000
D = 128
E = 320000

NS = 16
L = 16

CH = 128
NCHUNKS = E // CH
CHUNK_ITERS = (NCHUNKS + NS - 1) // NS

RFULL = NN // CH
RTAIL = NN - RFULL * CH
RITERS = (RFULL + NS - 1) // NS
NPAD = 10240

_f32 = jnp.float32
_i32 = jnp.int32


def _sc_body(item_emb, user_emb, gi_rows, gi_cols, gi_vals,
             gu_rows, gu_cols, gu_vals, a_rows, a_cols, a_vals,
             new_item, new_user, user_sums, item_sums,
             idx_buf, ridx_buf, val_buf, gbuf, ebuf, wbuf, redbuf, sbuf,
             acc, wrow, wcol, slab, sem):
    c = lax.axis_index("c")
    s = lax.axis_index("s")
    zeros16 = jnp.zeros((L,), _f32)

    def _zw(j, carry):
        wbuf[pl.ds(j * L, L)] = zeros16
        return carry
    lax.fori_loop(0, 640 // L, _zw, 0)
    pltpu.sync_copy(wbuf, wrow.at[pl.ds(s * 640, 640)])
    pltpu.sync_copy(wbuf, wcol.at[pl.ds(s * 640, 640)])
    plsc.subcore_barrier()

    def _w_pass(idx_hbm, w_ref):
        def body(k, carry):
            kk = s + k * NS
            @pl.when(kk < NCHUNKS)
            def _():
                e0 = kk * CH
                pltpu.sync_copy(idx_hbm.at[c, pl.ds(e0, CH)], idx_buf)
                pltpu.sync_copy(a_vals.at[c, pl.ds(e0, CH)], val_buf)
                pltpu.sync_copy(val_buf, w_ref.at[idx_buf], add=True)
            return carry
        lax.fori_loop(0, CHUNK_ITERS, body, 0)

    _w_pass(a_rows, wrow)
    _w_pass(a_cols, wcol)
    plsc.subcore_barrier()

    def _row_chunks(fn):
        def body(k, carry):
            m = s + k * NS
            @pl.when(m < RFULL)
            def _():
                fn(m * CH, CH)
            return carry
        lax.fori_loop(0, RITERS, body, 0)
        @pl.when(s == RFULL % NS)
        def _():
            fn(RFULL * CH, RTAIL)

    def _spmm(g, src_emb, dst_emb, rows_hbm, cols_hbm, vals_hbm, out_hbm):
        def _init(r0, n):
            pltpu.sync_copy(dst_emb.at[c, g, pl.ds(r0, n)], gbuf.at[pl.ds(0, n)])
            pltpu.sync_copy(gbuf.at[pl.ds(0, n)], acc.at[pl.ds(r0, n)])
        _row_chunks(_init)
        plsc.subcore_barrier()

        def body(k, carry):
            kk = s + k * NS
            @pl.when(kk < NCHUNKS)
            def _():
                e0 = kk * CH
                pltpu.sync_copy(cols_hbm.at[c, g, pl.ds(e0, CH)], idx_buf)
                pltpu.sync_copy(rows_hbm.at[c, g, pl.ds(e0, CH)], ridx_buf)
                pltpu.sync_copy(vals_hbm.at[c, g, pl.ds(e0, CH)], val_buf)
                pltpu.async_copy(src_emb.at[c, g].at[idx_buf], gbuf, sem).wait()

                def sgrp(jj, inner):
                    vv = val_buf[pl.ds(jj * L, L)]
                    for l in range(L):
                        v = vv[l]
                        row = jj * L + l
                        for t in range(D // L):
                            sl = pl.ds(t * L, L)
                            gbuf[row, sl] = gbuf[row, sl] * v
                    return inner
                lax.fori_loop(0, CH // L, sgrp, 0)
                pltpu.sync_copy(gbuf, acc.at[ridx_buf], add=True)
            return carry
        lax.fori_loop(0, CHUNK_ITERS, body, 0)
        plsc.subcore_barrier()

        def _wout(r0, n):
            pltpu.sync_copy(acc.at[pl.ds(r0, n)], gbuf.at[pl.ds(0, n)])
            pltpu.sync_copy(gbuf.at[pl.ds(0, n)], out_hbm.at[c, g, pl.ds(r0, n)])
        _row_chunks(_wout)

    def _reduce(g, dst_emb, w_ref, sum_out):
        for t in range(2 * D // L):
            redbuf[pl.ds(t * L, L)] = zeros16

        def _chunk(r0, n):
            pltpu.sync_copy(acc.at[pl.ds(r0, n)], gbuf.at[pl.ds(0, n)])
            pltpu.sync_copy(dst_emb.at[c, g, pl.ds(r0, n)], ebuf.at[pl.ds(0, n)])
            pltpu.sync_copy(w_ref.at[pl.ds(r0, n)], wbuf.at[pl.ds(0, n)])

            def rgrp(jj, carry):
                wv = wbuf[pl.ds(jj * L, L)]
                for l in range(L):
                    w = wv[l]
                    row = jj * L + l
                    for t in range(D // L):
                        sl = pl.ds(t * L, L)
                        plsc.addupdate(redbuf.at[sl], gbuf[row, sl] * w)
                        plsc.addupdate(redbuf.at[pl.ds(D + t * L, L)],
                                       ebuf[row, sl] * w)
                return carry
            lax.fori_loop(0, n // L, rgrp, 0)
        _row_chunks(_chunk)

        pltpu.sync_copy(redbuf, slab.at[s])
        plsc.subcore_barrier()

        @pl.when(s == 0)
        def _():
            pltpu.sync_copy(slab, sbuf)
            for t in range(D // L):
                sl = pl.ds(t * L, L)
                tot = zeros16
                for ss in range(NS):
                    tot = tot + (sbuf[ss, sl] - sbuf[ss, pl.ds(D + t * L, L)])
                val_buf[sl] = tot
            pltpu.sync_copy(val_buf, sum_out.at[c])

    for g in range(NG):
        _spmm(g, user_emb, item_emb, gu_rows, gu_cols, gu_vals, new_item)
        if g == NG - 1:
            _reduce(g, item_emb, wcol, user_sums)
        plsc.subcore_barrier()
        _spmm(g, item_emb, user_emb, gi_rows, gi_cols, gi_vals, new_user)
        if g == NG - 1:
            _reduce(g, user_emb, wrow, item_sums)
        plsc.subcore_barrier()


def _dense_body(usums, isums, uw, iw, out_u, out_i):
    ue = usums[...] * (1.0 / NU)
    ie = isums[...] * (1.0 / NI)
    mu = jnp.mean(ue, axis=0, keepdims=True)
    mi = jnp.mean(ie, axis=0, keepdims=True)
    pad = jnp.zeros((8 - NB - 1, D), _f32)
    xu = jnp.concatenate([ue, mu, pad], axis=0)
    xi = jnp.concatenate([ie, mi, pad], axis=0)
    out_u[...] = jax.nn.sigmoid(
        jnp.dot(xu, uw[...], preferred_element_type=_f32))
    out_i[...] = jax.nn.sigmoid(
        jnp.dot(xi, iw[...], preferred_element_type=_f32))


@functools.partial(jax.jit)
def kernel(item_embedding_g, user_embedding_g, gi_rows, gi_cols, gi_vals,
           gu_rows, gu_cols, gu_vals, A_rows, A_cols, A_vals, u_w, i_w):
    gi_rows = gi_rows.astype(_i32)
    gi_cols = gi_cols.astype(_i32)
    gu_rows = gu_rows.astype(_i32)
    gu_cols = gu_cols.astype(_i32)
    A_rows = A_rows.astype(_i32)
    A_cols = A_cols.astype(_i32)

    mesh = plsc.VectorSubcoreMesh(core_axis_name="c", subcore_axis_name="s")
    sc_call = pl.kernel(
        _sc_body,
        out_type=[
            jax.ShapeDtypeStruct((NB, NG, NI, D), _f32),
            jax.ShapeDtypeStruct((NB, NG, NU, D), _f32),
            jax.ShapeDtypeStruct((NB, D), _f32),
            jax.ShapeDtypeStruct((NB, D), _f32),
        ],
        mesh=mesh,
        scratch_types=[
            pltpu.VMEM((CH,), _i32),
            pltpu.VMEM((CH,), _i32),
            pltpu.VMEM((CH,), _f32),
            pltpu.VMEM((CH, D), _f32),
            pltpu.VMEM((CH, D), _f32),
            pltpu.VMEM((640,), _f32),
            pltpu.VMEM((2 * D,), _f32),
            pltpu.VMEM((NS, 2 * D), _f32),
            pltpu.VMEM_SHARED((NN, D), _f32),
            pltpu.VMEM_SHARED((NPAD,), _f32),
            pltpu.VMEM_SHARED((NPAD,), _f32),
            pltpu.VMEM_SHARED((NS, 2 * D), _f32),
            pltpu.SemaphoreType.DMA,
        ],
    )
    new_item_g, new_user_g, user_sums, item_sums = sc_call(
        item_embedding_g, user_embedding_g, gi_rows, gi_cols, gi_vals,
        gu_rows, gu_cols, gu_vals, A_rows, A_cols, A_vals)

    out_u, out_i = pl.pallas_call(
        _dense_body,
        out_shape=[jax.ShapeDtypeStruct((8, D), _f32),
                   jax.ShapeDtypeStruct((8, D), _f32)],
    )(user_sums, item_sums, u_w, i_w)

    user_embedding = out_u[NB]
    item_embedding = out_i[NB]
    user_embeddings_o = out_u[:NB]
    item_embeddings_o = out_i[:NB]
    return (user_embedding, item_embedding, user_embeddings_o,
            item_embeddings_o, new_item_g, new_user_g)


# speedup vs baseline: 7.6694x; 1.5328x over previous
import functools

import jax
import jax.numpy as jnp
from jax import lax
from jax.experimental import pallas as pl
from jax.experimental.pallas import tpu as pltpu
from jax.experimental.pallas import tpu_sc as plsc

NB = 2
NG = 2
NU = 10000
NI = 10000
NN = 10000
D = 128
E = 320000

NS = 16
L = 16

WCH = 128
NCHUNKS = E // WCH
CHUNK_ITERS = (NCHUNKS + NS - 1) // NS

CH = 64
CPB = 8
BPT = 40
CPT = CPB * BPT
ECH = CPT * NS
EPAD = ECH * CH

RFULL = NN // CH
RTAIL = NN - RFULL * CH
RITERS = (RFULL + NS - 1) // NS
NPAD = 10240

_f32 = jnp.float32
_i32 = jnp.int32


def _sc_body(item_emb, user_emb, gi_rows, gi_cols, gi_vals,
             gu_rows, gu_cols, gu_vals, a_rows, a_cols, a_vals,
             new_item, new_user, user_sums, item_sums,
             idx_buf, val_buf, cst, rst, vst_,
             gbuf0, gbuf1, scbuf0, scbuf1,
             wbuf, redbuf, sbuf,
             acc, wrow, wcol, slab,
             stsem, gsem0, gsem1, ssem0, ssem1):
    c = lax.axis_index("c")
    s = lax.axis_index("s")
    zeros16 = jnp.zeros((L,), _f32)
    gbufs = (gbuf0, gbuf1)
    gsems = (gsem0, gsem1)
    scbufs = (scbuf0, scbuf1)
    ssems = (ssem0, ssem1)
    ebuf = scbuf1

    for t in range(WCH // L):
        wbuf[pl.ds(t * L, L)] = zeros16

    def _zw(k, carry):
        w0 = (s + k * NS) * WCH
        pltpu.sync_copy(wbuf, wrow.at[pl.ds(w0, WCH)])
        pltpu.sync_copy(wbuf, wcol.at[pl.ds(w0, WCH)])
        return carry
    lax.fori_loop(0, NPAD // WCH // NS, _zw, 0)
    plsc.subcore_barrier()

    def _w_pass(idx_hbm, w_ref):
        def body(k, carry):
            kk = s + k * NS
            @pl.when(kk < NCHUNKS)
            def _():
                e0 = kk * WCH
                pltpu.sync_copy(idx_hbm.at[c, pl.ds(e0, WCH)], idx_buf)
                pltpu.sync_copy(a_vals.at[c, pl.ds(e0, WCH)], val_buf)
                pltpu.sync_copy(val_buf, w_ref.at[idx_buf], add=True)
            return carry
        lax.fori_loop(0, CHUNK_ITERS, body, 0)

    _w_pass(a_rows, wrow)
    _w_pass(a_cols, wcol)
    plsc.subcore_barrier()

    def _row_chunks(fn):
        def body(k, carry):
            m = s + k * NS
            @pl.when(m < RFULL)
            def _():
                fn(m * CH, CH)
            return carry
        lax.fori_loop(0, RITERS, body, 0)
        @pl.when(s == RFULL % NS)
        def _():
            fn(RFULL * CH, RTAIL)

    def _spmm(g, src_emb, dst_emb, rows2, cols2, vals2, out_hbm):
        def _init(r0, n):
            pltpu.sync_copy(dst_emb.at[c, g, pl.ds(r0, n)], gbuf0.at[pl.ds(0, n)])
            pltpu.sync_copy(gbuf0.at[pl.ds(0, n)], acc.at[pl.ds(r0, n)])
        _row_chunks(_init)
        plsc.subcore_barrier()

        start = s * CPT

        def blk(kb, carry):
            @pl.when(kb > 0)
            def _():
                for r in range(2):
                    pltpu.make_async_copy(
                        scbufs[r], acc.at[rst.at[r]], ssems[r]).wait()
            b0 = pl.multiple_of(start + kb * CPB, 8)
            d1 = pltpu.async_copy(cols2.at[c, g, pl.ds(b0, CPB)], cst, stsem)
            d2 = pltpu.async_copy(rows2.at[c, g, pl.ds(b0, CPB)], rst, stsem)
            d3 = pltpu.async_copy(vals2.at[c, g, pl.ds(b0, CPB)], vst_, stsem)
            d1.wait(); d2.wait(); d3.wait()

            pltpu.async_copy(
                src_emb.at[c, g].at[cst.at[0]], gbufs[0], gsems[0])

            def pair(p, inner):
                for h in range(2):
                    j = p * 2 + h
                    gb = gbufs[h]
                    sb = scbufs[h]
                    if h == 0:
                        pltpu.async_copy(
                            src_emb.at[c, g].at[cst.at[j + 1]],
                            gbufs[1], gsems[1])
                    else:
                        @pl.when(p < CPB // 2 - 1)
                        def _():
                            pltpu.async_copy(
                                src_emb.at[c, g].at[cst.at[j + 1]],
                                gbufs[0], gsems[0])
                    pltpu.make_async_copy(
                        src_emb.at[c, g].at[cst.at[j]], gb, gsems[h]).wait()
                    @pl.when(p > 0)
                    def _():
                        pltpu.make_async_copy(
                            sb, acc.at[rst.at[j]], ssems[h]).wait()

                    def sgrp(jj, sinner):
                        vv = vst_[j, pl.ds(jj * L, L)]
                        for l in range(L):
                            v = vv[l]
                            row = jj * L + l
                            for t in range(D // L):
                                sl = pl.ds(t * L, L)
                                sb[row, sl] = gb[row, sl] * v
                        return sinner
                    lax.fori_loop(0, CH // L, sgrp, 0)
                    pltpu.async_copy(
                        sb, acc.at[rst.at[j]], ssems[h], add=True)
                return inner
            lax.fori_loop(0, CPB // 2, pair, 0)
            return carry
        lax.fori_loop(0, BPT, blk, 0)
        for r in range(2):
            pltpu.make_async_copy(scbufs[r], acc.at[rst.at[r]],
                                  ssems[r]).wait()
        plsc.subcore_barrier()

        def _wout(r0, n):
            pltpu.sync_copy(acc.at[pl.ds(r0, n)], gbuf0.at[pl.ds(0, n)])
            pltpu.sync_copy(gbuf0.at[pl.ds(0, n)], out_hbm.at[c, g, pl.ds(r0, n)])
        _row_chunks(_wout)

    def _reduce(g, dst_emb, w_ref, sum_out):
        for t in range(2 * D // L):
            redbuf[pl.ds(t * L, L)] = zeros16

        def _chunk(r0, n):
            pltpu.sync_copy(acc.at[pl.ds(r0, n)], gbuf0.at[pl.ds(0, n)])
            pltpu.sync_copy(dst_emb.at[c, g, pl.ds(r0, n)], ebuf.at[pl.ds(0, n)])
            pltpu.sync_copy(w_ref.at[pl.ds(r0, n)], wbuf.at[pl.ds(0, n)])

            def rgrp(jj, carry):
                wv = wbuf[pl.ds(jj * L, L)]
                for l in range(L):
                    w = wv[l]
                    row = jj * L + l
                    for t in range(D // L):
                        sl = pl.ds(t * L, L)
                        plsc.addupdate(redbuf.at[sl], gbuf0[row, sl] * w)
                        plsc.addupdate(redbuf.at[pl.ds(D + t * L, L)],
                                       ebuf[row, sl] * w)
                return carry
            lax.fori_loop(0, n // L, rgrp, 0)
        _row_chunks(_chunk)

        pltpu.sync_copy(redbuf, slab.at[s])
        plsc.subcore_barrier()

        @pl.when(s == 0)
        def _():
            pltpu.sync_copy(slab, sbuf)
            for t in range(D // L):
                sl = pl.ds(t * L, L)
                tot = zeros16
                for ss in range(NS):
                    tot = tot + (sbuf[ss, sl] - sbuf[ss, pl.ds(D + t * L, L)])
                val_buf[sl] = tot
            pltpu.sync_copy(val_buf.at[pl.ds(0, D)], sum_out.at[c])

    def _side(src_emb, dst_emb, rows2, cols2, vals2, out_hbm, w_ref, sum_out):
        def gbody(g, carry):
            _spmm(g, src_emb, dst_emb, rows2, cols2, vals2, out_hbm)
            plsc.subcore_barrier()
            return carry
        lax.fori_loop(0, NG, gbody, 0)
        _reduce(NG - 1, dst_emb, w_ref, sum_out)
        plsc.subcore_barrier()

    _side(user_emb, item_emb, gu_rows, gu_cols, gu_vals, new_item,
          wcol, user_sums)
    _side(item_emb, user_emb, gi_rows, gi_cols, gi_vals, new_user,
          wrow, item_sums)


def _dense_body(usums, isums, uw, iw, out_u, out_i):
    ue = usums[...] * (1.0 / NU)
    ie = isums[...] * (1.0 / NI)
    mu = jnp.mean(ue, axis=0, keepdims=True)
    mi = jnp.mean(ie, axis=0, keepdims=True)
    pad = jnp.zeros((8 - NB - 1, D), _f32)
    xu = jnp.concatenate([ue, mu, pad], axis=0)
    xi = jnp.concatenate([ie, mi, pad], axis=0)
    out_u[...] = jax.nn.sigmoid(
        jnp.dot(xu, uw[...], preferred_element_type=_f32))
    out_i[...] = jax.nn.sigmoid(
        jnp.dot(xi, iw[...], preferred_element_type=_f32))


def _pad_edges(idx_or_val, pad_col):
    flat = idx_or_val.reshape(NB, NG, E)
    padv = jnp.broadcast_to(pad_col, (NB, NG, EPAD - E))
    return jnp.concatenate([flat, padv], axis=2).reshape(NB, NG, ECH, CH)


@functools.partial(jax.jit)
def kernel(item_embedding_g, user_embedding_g, gi_rows, gi_cols, gi_vals,
           gu_rows, gu_cols, gu_vals, A_rows, A_cols, A_vals, u_w, i_w):
    pad_idx = (jnp.arange(EPAD - E, dtype=_i32) * 37) % NN
    pad_val = jnp.zeros((EPAD - E,), _f32)
    gi_rows2 = _pad_edges(gi_rows.astype(_i32), pad_idx)
    gi_cols2 = _pad_edges(gi_cols.astype(_i32), pad_idx)
    gi_vals2 = _pad_edges(gi_vals, pad_val)
    gu_rows2 = _pad_edges(gu_rows.astype(_i32), pad_idx)
    gu_cols2 = _pad_edges(gu_cols.astype(_i32), pad_idx)
    gu_vals2 = _pad_edges(gu_vals, pad_val)
    A_rows = A_rows.astype(_i32)
    A_cols = A_cols.astype(_i32)

    mesh = plsc.VectorSubcoreMesh(core_axis_name="c", subcore_axis_name="s")
    sc_call = pl.kernel(
        _sc_body,
        out_type=[
            jax.ShapeDtypeStruct((NB, NG, NI, D), _f32),
            jax.ShapeDtypeStruct((NB, NG, NU, D), _f32),
            jax.ShapeDtypeStruct((NB, D), _f32),
            jax.ShapeDtypeStruct((NB, D), _f32),
        ],
        mesh=mesh,
        scratch_types=[
            pltpu.VMEM((WCH,), _i32),
            pltpu.VMEM((WCH,), _f32),
            pltpu.VMEM((CPB, CH), _i32),
            pltpu.VMEM((CPB, CH), _i32),
            pltpu.VMEM((CPB, CH), _f32),
            pltpu.VMEM((CH, D), _f32),
            pltpu.VMEM((CH, D), _f32),
            pltpu.VMEM((CH, D), _f32),
            pltpu.VMEM((CH, D), _f32),
            pltpu.VMEM((WCH,), _f32),
            pltpu.VMEM((2 * D,), _f32),
            pltpu.VMEM((NS, 2 * D), _f32),
            pltpu.VMEM_SHARED((NN, D), _f32),
            pltpu.VMEM_SHARED((NPAD,), _f32),
            pltpu.VMEM_SHARED((NPAD,), _f32),
            pltpu.VMEM_SHARED((NS, 2 * D), _f32),
            pltpu.SemaphoreType.DMA,
            pltpu.SemaphoreType.DMA,
            pltpu.SemaphoreType.DMA,
            pltpu.SemaphoreType.DMA,
            pltpu.SemaphoreType.DMA,
        ],
    )
    new_item_g, new_user_g, user_sums, item_sums = sc_call(
        item_embedding_g, user_embedding_g, gi_rows2, gi_cols2, gi_vals2,
        gu_rows2, gu_cols2, gu_vals2, A_rows, A_cols, A_vals)

    out_u, out_i = pl.pallas_call(
        _dense_body,
        out_shape=[jax.ShapeDtypeStruct((8, D), _f32),
                   jax.ShapeDtypeStruct((8, D), _f32)],
    )(user_sums, item_sums, u_w, i_w)

    user_embedding = out_u[NB]
    item_embedding = out_i[NB]
    user_embeddings_o = out_u[:NB]
    item_embeddings_o = out_i[:NB]
    return (user_embedding, item_embedding, user_embeddings_o,
            item_embeddings_o, new_item_g, new_user_g)


# speedup vs baseline: 8.8344x; 1.1519x over previous
import functools

import jax
import jax.numpy as jnp
from jax import lax
from jax.experimental import pallas as pl
from jax.experimental.pallas import tpu as pltpu
from jax.experimental.pallas import tpu_sc as plsc

NB = 2
NG = 2
NU = 10000
NI = 10000
NN = 10000
D = 128
E = 320000

NS = 16
L = 16

WCH = 128
WPB = 8
WBPT = 40
AECH = WPB * WBPT * NS
APAD = AECH * 64

CH = 64
CPB = 8
BPT = 40
CPT = CPB * BPT
ECH = CPT * NS
EPAD = ECH * CH

RFULL = NN // CH
RTAIL = NN - RFULL * CH
RITERS = (RFULL + NS - 1) // NS
NPAD = 10240

_f32 = jnp.float32
_i32 = jnp.int32


def _sc_body(item_emb, user_emb, gi_rows, gi_cols, gi_vals,
             gu_rows, gu_cols, gu_vals, a_rows, a_cols, a_vals,
             new_item, new_user, user_sums, item_sums,
             val_buf, cst, rst, vst_,
             gbuf0, gbuf1, scbuf0, scbuf1,
             wbuf, redbuf, sbuf,
             acc, wrow, wcol, slab,
             stsem, gsem0, gsem1, ssem0, ssem1, wsem):
    c = lax.axis_index("c")
    s = lax.axis_index("s")
    zeros16 = jnp.zeros((L,), _f32)
    gbufs = (gbuf0, gbuf1)
    gsems = (gsem0, gsem1)
    scbufs = (scbuf0, scbuf1)
    ssems = (ssem0, ssem1)
    ebuf = scbuf1

    for t in range(WCH // L):
        wbuf[pl.ds(t * L, L)] = zeros16

    def _zw(k, carry):
        w0 = (s + k * NS) * WCH
        pltpu.sync_copy(wbuf, wrow.at[pl.ds(w0, WCH)])
        pltpu.sync_copy(wbuf, wcol.at[pl.ds(w0, WCH)])
        return carry
    lax.fori_loop(0, NPAD // WCH // NS, _zw, 0)
    plsc.subcore_barrier()

    wstart = s * (WBPT * WPB)

    def _wblk(kb, carry):
        b0 = pl.multiple_of(wstart + kb * WPB, 8)
        d1 = pltpu.async_copy(a_rows.at[c, pl.ds(b0, WPB)], cst, stsem)
        d2 = pltpu.async_copy(a_cols.at[c, pl.ds(b0, WPB)], rst, stsem)
        d3 = pltpu.async_copy(a_vals.at[c, pl.ds(b0, WPB)], vst_, stsem)
        d1.wait(); d2.wait(); d3.wait()
        for j in range(WPB):
            pltpu.async_copy(vst_.at[j], wrow.at[cst.at[j]], wsem, add=True)
            pltpu.async_copy(vst_.at[j], wcol.at[rst.at[j]], wsem, add=True)
        for j in range(WPB):
            pltpu.make_async_copy(vst_.at[j], wrow.at[cst.at[j]], wsem).wait()
            pltpu.make_async_copy(vst_.at[j], wcol.at[rst.at[j]], wsem).wait()
        return carry
    lax.fori_loop(0, WBPT, _wblk, 0)
    plsc.subcore_barrier()

    def _row_chunks(fn):
        def body(k, carry):
            m = s + k * NS
            @pl.when(m < RFULL)
            def _():
                fn(m * CH, CH)
            return carry
        lax.fori_loop(0, RITERS, body, 0)
        @pl.when(s == RFULL % NS)
        def _():
            fn(RFULL * CH, RTAIL)

    def _spmm(g, src_emb, dst_emb, rows2, cols2, vals2, out_hbm):
        def _init(r0, n):
            pltpu.sync_copy(dst_emb.at[c, g, pl.ds(r0, n)], acc.at[pl.ds(r0, n)])
        _row_chunks(_init)
        plsc.subcore_barrier()

        start = s * CPT

        def blk(kb, carry):
            @pl.when(kb > 0)
            def _():
                for r in range(2):
                    pltpu.make_async_copy(
                        scbufs[r], acc.at[rst.at[r]], ssems[r]).wait()
            b0 = pl.multiple_of(start + kb * CPB, 8)
            d1 = pltpu.async_copy(cols2.at[c, g, pl.ds(b0, CPB)], cst, stsem)
            d2 = pltpu.async_copy(rows2.at[c, g, pl.ds(b0, CPB)], rst, stsem)
            d3 = pltpu.async_copy(vals2.at[c, g, pl.ds(b0, CPB)], vst_, stsem)
            d1.wait(); d2.wait(); d3.wait()

            pltpu.async_copy(
                src_emb.at[c, g].at[cst.at[0]], gbufs[0], gsems[0])

            def pair(p, inner):
                for h in range(2):
                    j = p * 2 + h
                    gb = gbufs[h]
                    sb = scbufs[h]
                    if h == 0:
                        pltpu.async_copy(
                            src_emb.at[c, g].at[cst.at[j + 1]],
                            gbufs[1], gsems[1])
                    else:
                        @pl.when(p < CPB // 2 - 1)
                        def _():
                            pltpu.async_copy(
                                src_emb.at[c, g].at[cst.at[j + 1]],
                                gbufs[0], gsems[0])
                    pltpu.make_async_copy(
                        src_emb.at[c, g].at[cst.at[j]], gb, gsems[h]).wait()
                    @pl.when(p > 0)
                    def _():
                        pltpu.make_async_copy(
                            sb, acc.at[rst.at[j]], ssems[h]).wait()

                    def sgrp(jj, sinner):
                        vv = vst_[j, pl.ds(jj * L, L)]
                        for l in range(L):
                            v = vv[l]
                            row = jj * L + l
                            for t in range(D // L):
                                sl = pl.ds(t * L, L)
                                sb[row, sl] = gb[row, sl] * v
                        return sinner
                    lax.fori_loop(0, CH // L, sgrp, 0)
                    pltpu.async_copy(
                        sb, acc.at[rst.at[j]], ssems[h], add=True)
                return inner
            lax.fori_loop(0, CPB // 2, pair, 0)
            return carry
        lax.fori_loop(0, BPT, blk, 0)
        for r in range(2):
            pltpu.make_async_copy(scbufs[r], acc.at[rst.at[r]],
                                  ssems[r]).wait()
        plsc.subcore_barrier()

        def _wout(r0, n):
            pltpu.sync_copy(acc.at[pl.ds(r0, n)], out_hbm.at[c, g, pl.ds(r0, n)])
        _row_chunks(_wout)

    def _reduce(g, dst_emb, w_ref, sum_out):
        for t in range(2 * D // L):
            redbuf[pl.ds(t * L, L)] = zeros16

        def _chunk(r0, n):
            pltpu.sync_copy(acc.at[pl.ds(r0, n)], gbuf0.at[pl.ds(0, n)])
            pltpu.sync_copy(dst_emb.at[c, g, pl.ds(r0, n)], ebuf.at[pl.ds(0, n)])
            pltpu.sync_copy(w_ref.at[pl.ds(r0, n)], wbuf.at[pl.ds(0, n)])

            def rgrp(jj, carry):
                wv = wbuf[pl.ds(jj * L, L)]
                for l in range(L):
                    w = wv[l]
                    row = jj * L + l
                    for t in range(D // L):
                        sl = pl.ds(t * L, L)
                        plsc.addupdate(redbuf.at[sl], gbuf0[row, sl] * w)
                        plsc.addupdate(redbuf.at[pl.ds(D + t * L, L)],
                                       ebuf[row, sl] * w)
                return carry
            lax.fori_loop(0, n // L, rgrp, 0)
        _row_chunks(_chunk)

        pltpu.sync_copy(redbuf, slab.at[s])
        plsc.subcore_barrier()

        @pl.when(s == 0)
        def _():
            pltpu.sync_copy(slab, sbuf)
            for t in range(D // L):
                sl = pl.ds(t * L, L)
                tot = zeros16
                for ss in range(NS):
                    tot = tot + (sbuf[ss, sl] - sbuf[ss, pl.ds(D + t * L, L)])
                val_buf[sl] = tot
            pltpu.sync_copy(val_buf.at[pl.ds(0, D)], sum_out.at[c])

    def _side(src_emb, dst_emb, rows2, cols2, vals2, out_hbm, w_ref, sum_out):
        def gbody(g, carry):
            _spmm(g, src_emb, dst_emb, rows2, cols2, vals2, out_hbm)
            plsc.subcore_barrier()
            return carry
        lax.fori_loop(0, NG, gbody, 0)
        _reduce(NG - 1, dst_emb, w_ref, sum_out)
        plsc.subcore_barrier()

    _side(user_emb, item_emb, gu_rows, gu_cols, gu_vals, new_item,
          wcol, user_sums)
    _side(item_emb, user_emb, gi_rows, gi_cols, gi_vals, new_user,
          wrow, item_sums)


def _dense_body(usums, isums, uw, iw, out_u, out_i):
    ue = usums[...] * (1.0 / NU)
    ie = isums[...] * (1.0 / NI)
    mu = jnp.mean(ue, axis=0, keepdims=True)
    mi = jnp.mean(ie, axis=0, keepdims=True)
    pad = jnp.zeros((8 - NB - 1, D), _f32)
    xu = jnp.concatenate([ue, mu, pad], axis=0)
    xi = jnp.concatenate([ie, mi, pad], axis=0)
    out_u[...] = jax.nn.sigmoid(
        jnp.dot(xu, uw[...], preferred_element_type=_f32))
    out_i[...] = jax.nn.sigmoid(
        jnp.dot(xi, iw[...], preferred_element_type=_f32))


def _pad_edges(idx_or_val, pad_col):
    flat = idx_or_val.reshape(NB, NG, E)
    padv = jnp.broadcast_to(pad_col, (NB, NG, EPAD - E))
    return jnp.concatenate([flat, padv], axis=2).reshape(NB, NG, ECH, CH)


@functools.partial(jax.jit)
def kernel(item_embedding_g, user_embedding_g, gi_rows, gi_cols, gi_vals,
           gu_rows, gu_cols, gu_vals, A_rows, A_cols, A_vals, u_w, i_w):
    pad_idx = (jnp.arange(EPAD - E, dtype=_i32) * 37) % NN
    pad_val = jnp.zeros((EPAD - E,), _f32)
    gi_rows2 = _pad_edges(gi_rows.astype(_i32), pad_idx)
    gi_cols2 = _pad_edges(gi_cols.astype(_i32), pad_idx)
    gi_vals2 = _pad_edges(gi_vals, pad_val)
    gu_rows2 = _pad_edges(gu_rows.astype(_i32), pad_idx)
    gu_cols2 = _pad_edges(gu_cols.astype(_i32), pad_idx)
    gu_vals2 = _pad_edges(gu_vals, pad_val)
    apad_idx = (jnp.arange(APAD - E, dtype=_i32) * 53) % NN
    apad_val = jnp.zeros((APAD - E,), _f32)

    def _pad_a(x, padv):
        padv = jnp.broadcast_to(padv, (NB, APAD - E))
        return jnp.concatenate([x, padv], axis=1).reshape(NB, AECH, 64)

    A_rows2 = _pad_a(A_rows.astype(_i32), apad_idx)
    A_cols2 = _pad_a(A_cols.astype(_i32), apad_idx)
    A_vals2 = _pad_a(A_vals, apad_val)

    mesh = plsc.VectorSubcoreMesh(core_axis_name="c", subcore_axis_name="s")
    sc_call = pl.kernel(
        _sc_body,
        out_type=[
            jax.ShapeDtypeStruct((NB, NG, NI, D), _f32),
            jax.ShapeDtypeStruct((NB, NG, NU, D), _f32),
            jax.ShapeDtypeStruct((NB, D), _f32),
            jax.ShapeDtypeStruct((NB, D), _f32),
        ],
        mesh=mesh,
        scratch_types=[
            pltpu.VMEM((WCH,), _f32),
            pltpu.VMEM((CPB, CH), _i32),
            pltpu.VMEM((CPB, CH), _i32),
            pltpu.VMEM((CPB, CH), _f32),
            pltpu.VMEM((CH, D), _f32),
            pltpu.VMEM((CH, D), _f32),
            pltpu.VMEM((CH, D), _f32),
            pltpu.VMEM((CH, D), _f32),
            pltpu.VMEM((WCH,), _f32),
            pltpu.VMEM((2 * D,), _f32),
            pltpu.VMEM((NS, 2 * D), _f32),
            pltpu.VMEM_SHARED((NN, D), _f32),
            pltpu.VMEM_SHARED((NPAD,), _f32),
            pltpu.VMEM_SHARED((NPAD,), _f32),
            pltpu.VMEM_SHARED((NS, 2 * D), _f32),
            pltpu.SemaphoreType.DMA,
            pltpu.SemaphoreType.DMA,
            pltpu.SemaphoreType.DMA,
            pltpu.SemaphoreType.DMA,
            pltpu.SemaphoreType.DMA,
            pltpu.SemaphoreType.DMA,
        ],
    )
    new_item_g, new_user_g, user_sums, item_sums = sc_call(
        item_embedding_g, user_embedding_g, gi_rows2, gi_cols2, gi_vals2,
        gu_rows2, gu_cols2, gu_vals2, A_rows2, A_cols2, A_vals2)

    out_u, out_i = pl.pallas_call(
        _dense_body,
        out_shape=[jax.ShapeDtypeStruct((8, D), _f32),
                   jax.ShapeDtypeStruct((8, D), _f32)],
    )(user_sums, item_sums, u_w, i_w)

    user_embedding = out_u[NB]
    item_embedding = out_i[NB]
    user_embeddings_o = out_u[:NB]
    item_embeddings_o = out_i[:NB]
    return (user_embedding, item_embedding, user_embeddings_o,
            item_embeddings_o, new_item_g, new_user_g)


# speedup vs baseline: 11.2164x; 1.2696x over previous
import functools

import jax
import jax.numpy as jnp
from jax import lax
from jax.experimental import pallas as pl
from jax.experimental.pallas import tpu as pltpu
from jax.experimental.pallas import tpu_sc as plsc

NB = 2
NG = 2
NU = 10000
NI = 10000
NN = 10000
D = 128
E = 320000

NS = 16
L = 16

WCH = 128
WPB = 8
WBPT = 40
AECH = WPB * WBPT * NS
APAD = AECH * 64

CH = 64
CPB = 8
BPT = 40
CPT = CPB * BPT
ECH = CPT * NS
EPAD = ECH * CH

RFULL = NN // CH
RTAIL = NN - RFULL * CH
RITERS = (RFULL + NS - 1) // NS
NPAD = 10240

_f32 = jnp.float32
_i32 = jnp.int32


def _sc_body(item_emb, user_emb, gi_rows, gi_cols, gi_vals,
             gu_rows, gu_cols, gu_vals, a_rows, a_cols, a_vals,
             new_item, new_user, user_sums, item_sums,
             val_buf, cst3, rst3, vst3,
             buf0, buf1, buf2, buf3,
             wbuf, redbuf, sbuf,
             acc, wrow, wcol, slab,
             stsem, sem0, sem1, sem2, sem3, wsem):
    c = lax.axis_index("c")
    s = lax.axis_index("s")
    zeros16 = jnp.zeros((L,), _f32)
    bufs = (buf0, buf1, buf2, buf3)
    sems = (sem0, sem1, sem2, sem3)
    gbuf0 = buf0
    ebuf = buf1

    for t in range(WCH // L):
        wbuf[pl.ds(t * L, L)] = zeros16

    def _zw(k, carry):
        w0 = (s + k * NS) * WCH
        pltpu.sync_copy(wbuf, wrow.at[pl.ds(w0, WCH)])
        pltpu.sync_copy(wbuf, wcol.at[pl.ds(w0, WCH)])
        return carry
    lax.fori_loop(0, NPAD // WCH // NS, _zw, 0)
    plsc.subcore_barrier()

    wstart = s * (WBPT * WPB)

    def _wblk(kb, carry):
        b0 = pl.multiple_of(wstart + kb * WPB, 8)
        d1 = pltpu.async_copy(a_rows.at[c, pl.ds(b0, WPB)], cst3.at[0], stsem)
        d2 = pltpu.async_copy(a_cols.at[c, pl.ds(b0, WPB)], rst3.at[0], stsem)
        d3 = pltpu.async_copy(a_vals.at[c, pl.ds(b0, WPB)], vst3.at[0], stsem)
        d1.wait(); d2.wait(); d3.wait()
        for j in range(WPB):
            pltpu.async_copy(vst3.at[0, j], wrow.at[cst3.at[0, j]], wsem,
                             add=True)
            pltpu.async_copy(vst3.at[0, j], wcol.at[rst3.at[0, j]], wsem,
                             add=True)
        for j in range(WPB):
            pltpu.make_async_copy(vst3.at[0, j], wrow.at[cst3.at[0, j]],
                                  wsem).wait()
            pltpu.make_async_copy(vst3.at[0, j], wcol.at[rst3.at[0, j]],
                                  wsem).wait()
        return carry
    lax.fori_loop(0, WBPT, _wblk, 0)
    plsc.subcore_barrier()

    def _row_chunks(fn):
        def body(k, carry):
            m = s + k * NS
            @pl.when(m < RFULL)
            def _():
                fn(m * CH, CH)
            return carry
        lax.fori_loop(0, RITERS, body, 0)
        @pl.when(s == RFULL % NS)
        def _():
            fn(RFULL * CH, RTAIL)

    def _spmm(g, src_emb, dst_emb, rows2, cols2, vals2, out_hbm):
        def _init(r0, n):
            pltpu.sync_copy(dst_emb.at[c, g, pl.ds(r0, n)], acc.at[pl.ds(r0, n)])
        _row_chunks(_init)
        plsc.subcore_barrier()

        start = s * CPT

        def _stage(kbx, par):
            b0 = pl.multiple_of(start + kbx * CPB, 8)
            pltpu.async_copy(cols2.at[c, g, pl.ds(b0, CPB)], cst3.at[par],
                             stsem)
            pltpu.async_copy(rows2.at[c, g, pl.ds(b0, CPB)], rst3.at[par],
                             stsem)
            pltpu.async_copy(vals2.at[c, g, pl.ds(b0, CPB)], vst3.at[par],
                             stsem)

        def _stage_wait(par):
            for ref, arr in ((cst3, cols2), (rst3, rows2), (vst3, vals2)):
                pltpu.make_async_copy(arr.at[c, g, pl.ds(0, CPB)],
                                      ref.at[par], stsem).wait()

        def _gissue(par, row, slot):
            pltpu.async_copy(src_emb.at[c, g].at[cst3.at[par, row]],
                             bufs[slot], sems[slot])

        def _gwait(par, row, slot):
            pltpu.make_async_copy(src_emb.at[c, g].at[cst3.at[par, row]],
                                  bufs[slot], sems[slot]).wait()

        def _sissue(par, row, slot):
            pltpu.async_copy(bufs[slot], acc.at[rst3.at[par, row]],
                             sems[slot], add=True)

        def _sdrain(par, row, slot):
            pltpu.make_async_copy(bufs[slot], acc.at[rst3.at[par, row]],
                                  sems[slot]).wait()

        _stage(0, 0)

        def blk(kb, carry):
            hh = lax.rem(kb, 2)
            _stage_wait(hh)
            @pl.when(kb < BPT - 1)
            def _():
                _stage(kb + 1, 1 - hh)
            _gissue(hh, 0, 0)
            _gissue(hh, 1, 1)

            def quad(q, inner):
                for i in range(4):
                    j = q * 4 + i
                    slot = i
                    nslot = (i + 2) % 4
                    if i < 2:
                        @pl.when((kb > 0) | (q > 0))
                        def _():
                            _sdrain(hh, j, nslot)
                    else:
                        _sdrain(hh, j, nslot)
                    if i < 2:
                        _gissue(hh, j + 2, nslot)
                    else:
                        @pl.when(q == 0)
                        def _():
                            _gissue(hh, j + 2, nslot)
                    _gwait(hh, j, slot)
                    gb = bufs[slot]

                    def sgrp(jj, sinner):
                        vv = vst3[hh, j, pl.ds(jj * L, L)]
                        for l in range(L):
                            v = vv[l]
                            row = jj * L + l
                            for t in range(D // L):
                                sl = pl.ds(t * L, L)
                                gb[row, sl] = gb[row, sl] * v
                        return sinner
                    lax.fori_loop(0, CH // L, sgrp, 0)
                    _sissue(hh, j, slot)
                return inner
            lax.fori_loop(0, CPB // 4, quad, 0)
            return carry
        lax.fori_loop(0, BPT, blk, 0)
        hlast = (BPT - 1) % 2
        _sdrain(hlast, 6, 2)
        _sdrain(hlast, 7, 3)
        plsc.subcore_barrier()

        def _wout(r0, n):
            pltpu.sync_copy(acc.at[pl.ds(r0, n)], out_hbm.at[c, g, pl.ds(r0, n)])
        _row_chunks(_wout)

    def _reduce(g, dst_emb, w_ref, sum_out):
        for t in range(2 * D // L):
            redbuf[pl.ds(t * L, L)] = zeros16

        def _chunk(r0, n):
            pltpu.sync_copy(acc.at[pl.ds(r0, n)], gbuf0.at[pl.ds(0, n)])
            pltpu.sync_copy(dst_emb.at[c, g, pl.ds(r0, n)], ebuf.at[pl.ds(0, n)])
            pltpu.sync_copy(w_ref.at[pl.ds(r0, n)], wbuf.at[pl.ds(0, n)])

            def rgrp(jj, carry):
                wv = wbuf[pl.ds(jj * L, L)]
                for l in range(L):
                    w = wv[l]
                    row = jj * L + l
                    for t in range(D // L):
                        sl = pl.ds(t * L, L)
                        plsc.addupdate(redbuf.at[sl], gbuf0[row, sl] * w)
                        plsc.addupdate(redbuf.at[pl.ds(D + t * L, L)],
                                       ebuf[row, sl] * w)
                return carry
            lax.fori_loop(0, n // L, rgrp, 0)
        _row_chunks(_chunk)

        pltpu.sync_copy(redbuf, slab.at[s])
        plsc.subcore_barrier()

        @pl.when(s == 0)
        def _():
            pltpu.sync_copy(slab, sbuf)
            for t in range(D // L):
                sl = pl.ds(t * L, L)
                tot = zeros16
                for ss in range(NS):
                    tot = tot + (sbuf[ss, sl] - sbuf[ss, pl.ds(D + t * L, L)])
                val_buf[sl] = tot
            pltpu.sync_copy(val_buf.at[pl.ds(0, D)], sum_out.at[c])

    def _side(src_emb, dst_emb, rows2, cols2, vals2, out_hbm, w_ref, sum_out):
        def gbody(g, carry):
            _spmm(g, src_emb, dst_emb, rows2, cols2, vals2, out_hbm)
            plsc.subcore_barrier()
            return carry
        lax.fori_loop(0, NG, gbody, 0)
        _reduce(NG - 1, dst_emb, w_ref, sum_out)
        plsc.subcore_barrier()

    _side(user_emb, item_emb, gu_rows, gu_cols, gu_vals, new_item,
          wcol, user_sums)
    _side(item_emb, user_emb, gi_rows, gi_cols, gi_vals, new_user,
          wrow, item_sums)


def _dense_body(usums, isums, uw, iw, out_u, out_i):
    ue = usums[...] * (1.0 / NU)
    ie = isums[...] * (1.0 / NI)
    mu = jnp.mean(ue, axis=0, keepdims=True)
    mi = jnp.mean(ie, axis=0, keepdims=True)
    pad = jnp.zeros((8 - NB - 1, D), _f32)
    xu = jnp.concatenate([ue, mu, pad], axis=0)
    xi = jnp.concatenate([ie, mi, pad], axis=0)
    out_u[...] = jax.nn.sigmoid(
        jnp.dot(xu, uw[...], preferred_element_type=_f32))
    out_i[...] = jax.nn.sigmoid(
        jnp.dot(xi, iw[...], preferred_element_type=_f32))


def _pad_edges(idx_or_val, pad_col):
    flat = idx_or_val.reshape(NB, NG, E)
    padv = jnp.broadcast_to(pad_col, (NB, NG, EPAD - E))
    return jnp.concatenate([flat, padv], axis=2).reshape(NB, NG, ECH, CH)


@functools.partial(jax.jit)
def kernel(item_embedding_g, user_embedding_g, gi_rows, gi_cols, gi_vals,
           gu_rows, gu_cols, gu_vals, A_rows, A_cols, A_vals, u_w, i_w):
    pad_idx = (jnp.arange(EPAD - E, dtype=_i32) * 37) % NN
    pad_val = jnp.zeros((EPAD - E,), _f32)
    gi_rows2 = _pad_edges(gi_rows.astype(_i32), pad_idx)
    gi_cols2 = _pad_edges(gi_cols.astype(_i32), pad_idx)
    gi_vals2 = _pad_edges(gi_vals, pad_val)
    gu_rows2 = _pad_edges(gu_rows.astype(_i32), pad_idx)
    gu_cols2 = _pad_edges(gu_cols.astype(_i32), pad_idx)
    gu_vals2 = _pad_edges(gu_vals, pad_val)
    apad_idx = (jnp.arange(APAD - E, dtype=_i32) * 53) % NN
    apad_val = jnp.zeros((APAD - E,), _f32)

    def _pad_a(x, padv):
        padv = jnp.broadcast_to(padv, (NB, APAD - E))
        return jnp.concatenate([x, padv], axis=1).reshape(NB, AECH, 64)

    A_rows2 = _pad_a(A_rows.astype(_i32), apad_idx)
    A_cols2 = _pad_a(A_cols.astype(_i32), apad_idx)
    A_vals2 = _pad_a(A_vals, apad_val)

    mesh = plsc.VectorSubcoreMesh(core_axis_name="c", subcore_axis_name="s")
    sc_call = pl.kernel(
        _sc_body,
        out_type=[
            jax.ShapeDtypeStruct((NB, NG, NI, D), _f32),
            jax.ShapeDtypeStruct((NB, NG, NU, D), _f32),
            jax.ShapeDtypeStruct((NB, D), _f32),
            jax.ShapeDtypeStruct((NB, D), _f32),
        ],
        mesh=mesh,
        scratch_types=[
            pltpu.VMEM((WCH,), _f32),
            pltpu.VMEM((2, CPB, CH), _i32),
            pltpu.VMEM((2, CPB, CH), _i32),
            pltpu.VMEM((2, CPB, CH), _f32),
            pltpu.VMEM((CH, D), _f32),
            pltpu.VMEM((CH, D), _f32),
            pltpu.VMEM((CH, D), _f32),
            pltpu.VMEM((CH, D), _f32),
            pltpu.VMEM((WCH,), _f32),
            pltpu.VMEM((2 * D,), _f32),
            pltpu.VMEM((NS, 2 * D), _f32),
            pltpu.VMEM_SHARED((NN, D), _f32),
            pltpu.VMEM_SHARED((NPAD,), _f32),
            pltpu.VMEM_SHARED((NPAD,), _f32),
            pltpu.VMEM_SHARED((NS, 2 * D), _f32),
            pltpu.SemaphoreType.DMA,
            pltpu.SemaphoreType.DMA,
            pltpu.SemaphoreType.DMA,
            pltpu.SemaphoreType.DMA,
            pltpu.SemaphoreType.DMA,
            pltpu.SemaphoreType.DMA,
        ],
    )
    new_item_g, new_user_g, user_sums, item_sums = sc_call(
        item_embedding_g, user_embedding_g, gi_rows2, gi_cols2, gi_vals2,
        gu_rows2, gu_cols2, gu_vals2, A_rows2, A_cols2, A_vals2)

    out_u, out_i = pl.pallas_call(
        _dense_body,
        out_shape=[jax.ShapeDtypeStruct((8, D), _f32),
                   jax.ShapeDtypeStruct((8, D), _f32)],
    )(user_sums, item_sums, u_w, i_w)

    user_embedding = out_u[NB]
    item_embedding = out_i[NB]
    user_embeddings_o = out_u[:NB]
    item_embeddings_o = out_i[:NB]
    return (user_embedding, item_embedding, user_embeddings_o,
            item_embeddings_o, new_item_g, new_user_g)


# speedup vs baseline: 12.6474x; 1.1276x over previous
import functools

import jax
import jax.numpy as jnp
from jax import lax
from jax.experimental import pallas as pl
from jax.experimental.pallas import tpu as pltpu
from jax.experimental.pallas import tpu_sc as plsc

NB = 2
NG = 2
NU = 10000
NI = 10000
NN = 10000
D = 128
E = 320000

NS = 16
L = 16

WCH = 128
WPB = 8
WBPT = 40
AECH = WPB * WBPT * NS
APAD = AECH * 64

CH = 64
CPB = 8
BPT = 40
CPT = CPB * BPT
ECH = CPT * NS
EPAD = ECH * CH

RFULL = NN // CH
RTAIL = NN - RFULL * CH
RITERS = (RFULL + NS - 1) // NS
NPAD = 10240

_f32 = jnp.float32
_i32 = jnp.int32


def _sc_body(item_emb, user_emb, gi_rows, gi_cols, gi_vals,
             gu_rows, gu_cols, gu_vals, a_rows, a_cols, a_vals,
             new_item, new_user, user_sums, item_sums,
             val_buf, cst3, rst3, vst3,
             buf0, buf1, buf2, buf3,
             wbuf, redbuf, sbuf,
             acc, wrow, wcol, slab,
             stsem, sem0, sem1, sem2, sem3, wsem):
    c = lax.axis_index("c")
    s = lax.axis_index("s")
    zeros16 = jnp.zeros((L,), _f32)
    bufs = (buf0, buf1, buf2, buf3)
    sems = (sem0, sem1, sem2, sem3)
    gbuf0 = buf0
    ebuf = buf1

    for t in range(WCH // L):
        wbuf[pl.ds(t * L, L)] = zeros16

    def _zw(k, carry):
        w0 = (s + k * NS) * WCH
        pltpu.sync_copy(wbuf, wrow.at[pl.ds(w0, WCH)])
        pltpu.sync_copy(wbuf, wcol.at[pl.ds(w0, WCH)])
        return carry
    lax.fori_loop(0, NPAD // WCH // NS, _zw, 0)
    plsc.subcore_barrier()

    wstart = s * (WBPT * WPB)

    def _wblk(kb, carry):
        b0 = pl.multiple_of(wstart + kb * WPB, 8)
        d1 = pltpu.async_copy(a_rows.at[c, pl.ds(b0, WPB)], cst3.at[0], stsem)
        d2 = pltpu.async_copy(a_cols.at[c, pl.ds(b0, WPB)], rst3.at[0], stsem)
        d3 = pltpu.async_copy(a_vals.at[c, pl.ds(b0, WPB)], vst3.at[0], stsem)
        d1.wait(); d2.wait(); d3.wait()
        for j in range(WPB):
            pltpu.async_copy(vst3.at[0, j], wrow.at[cst3.at[0, j]], wsem,
                             add=True)
            pltpu.async_copy(vst3.at[0, j], wcol.at[rst3.at[0, j]], wsem,
                             add=True)
        for j in range(WPB):
            pltpu.make_async_copy(vst3.at[0, j], wrow.at[cst3.at[0, j]],
                                  wsem).wait()
            pltpu.make_async_copy(vst3.at[0, j], wcol.at[rst3.at[0, j]],
                                  wsem).wait()
        return carry
    lax.fori_loop(0, WBPT, _wblk, 0)
    plsc.subcore_barrier()

    def _row_chunks(fn):
        def body(k, carry):
            m = s + k * NS
            @pl.when(m < RFULL)
            def _():
                fn(m * CH, CH)
            return carry
        lax.fori_loop(0, RITERS, body, 0)
        @pl.when(s == RFULL % NS)
        def _():
            fn(RFULL * CH, RTAIL)

    def _spmm(g, src_emb, dst_emb, rows2, cols2, vals2, out_hbm):
        def _init(r0, n):
            pltpu.sync_copy(dst_emb.at[c, g, pl.ds(r0, n)], acc.at[pl.ds(r0, n)])
        _row_chunks(_init)
        plsc.subcore_barrier()

        start = s * CPT

        def _stage(kbx, par):
            b0 = pl.multiple_of(start + kbx * CPB, 8)
            pltpu.async_copy(cols2.at[c, g, pl.ds(b0, CPB)], cst3.at[par],
                             stsem)
            pltpu.async_copy(rows2.at[c, g, pl.ds(b0, CPB)], rst3.at[par],
                             stsem)
            pltpu.async_copy(vals2.at[c, g, pl.ds(b0, CPB)], vst3.at[par],
                             stsem)

        def _stage_wait(par):
            for ref, arr in ((cst3, cols2), (rst3, rows2), (vst3, vals2)):
                pltpu.make_async_copy(arr.at[c, g, pl.ds(0, CPB)],
                                      ref.at[par], stsem).wait()

        def _gissue(par, row, slot):
            pltpu.async_copy(src_emb.at[c, g].at[cst3.at[par, row]],
                             bufs[slot], sems[slot])

        def _gwait(par, row, slot):
            pltpu.make_async_copy(src_emb.at[c, g].at[cst3.at[par, row]],
                                  bufs[slot], sems[slot]).wait()

        def _sissue(par, row, slot):
            pltpu.async_copy(bufs[slot], acc.at[rst3.at[par, row]],
                             sems[slot], add=True)

        def _sdrain(par, row, slot):
            pltpu.make_async_copy(bufs[slot], acc.at[rst3.at[par, row]],
                                  sems[slot]).wait()

        _stage(0, 0)
        _stage_wait(0)
        _gissue(0, 0, 0)
        _gissue(0, 1, 1)

        def blk(kb, carry):
            hh = lax.rem(kb, 2)

            def quad(q, inner):
                for i in range(4):
                    j = q * 4 + i
                    slot = i
                    nslot = (i + 2) % 4
                    if i < 2:
                        @pl.when((kb > 0) | (q > 0))
                        def _():
                            _sdrain(hh, j, nslot)
                    else:
                        _sdrain(hh, j, nslot)
                    if i < 2:
                        _gissue(hh, j + 2, nslot)
                        if i == 1:
                            @pl.when((q == 0) & (kb < BPT - 1))
                            def _():
                                _stage(kb + 1, 1 - hh)
                    else:
                        @pl.when(q == 0)
                        def _():
                            _gissue(hh, j + 2, nslot)
                        if i == 2:
                            @pl.when((q == 1) & (kb < BPT - 1))
                            def _():
                                _stage_wait(1 - hh)
                                _gissue(1 - hh, 0, 0)
                        else:
                            @pl.when((q == 1) & (kb < BPT - 1))
                            def _():
                                _gissue(1 - hh, 1, 1)
                    _gwait(hh, j, slot)
                    gb = bufs[slot]

                    def sgrp(jj, sinner):
                        vv = vst3[hh, j, pl.ds(jj * L, L)]
                        for l in range(L):
                            v = vv[l]
                            row = jj * L + l
                            for t in range(D // L):
                                sl = pl.ds(t * L, L)
                                gb[row, sl] = gb[row, sl] * v
                        return sinner
                    lax.fori_loop(0, CH // L, sgrp, 0)
                    _sissue(hh, j, slot)
                return inner
            lax.fori_loop(0, CPB // 4, quad, 0)
            return carry
        lax.fori_loop(0, BPT, blk, 0)
        hlast = (BPT - 1) % 2
        _sdrain(hlast, 6, 2)
        _sdrain(hlast, 7, 3)
        plsc.subcore_barrier()

        def _wout(r0, n):
            pltpu.sync_copy(acc.at[pl.ds(r0, n)], out_hbm.at[c, g, pl.ds(r0, n)])
        _row_chunks(_wout)

    def _reduce(g, dst_emb, w_ref, sum_out):
        for t in range(2 * D // L):
            redbuf[pl.ds(t * L, L)] = zeros16

        def _chunk(r0, n):
            pltpu.sync_copy(acc.at[pl.ds(r0, n)], gbuf0.at[pl.ds(0, n)])
            pltpu.sync_copy(dst_emb.at[c, g, pl.ds(r0, n)], ebuf.at[pl.ds(0, n)])
            pltpu.sync_copy(w_ref.at[pl.ds(r0, n)], wbuf.at[pl.ds(0, n)])

            def rgrp(jj, carry):
                wv = wbuf[pl.ds(jj * L, L)]
                for l in range(L):
                    w = wv[l]
                    row = jj * L + l
                    for t in range(D // L):
                        sl = pl.ds(t * L, L)
                        plsc.addupdate(redbuf.at[sl], gbuf0[row, sl] * w)
                        plsc.addupdate(redbuf.at[pl.ds(D + t * L, L)],
                                       ebuf[row, sl] * w)
                return carry
            lax.fori_loop(0, n // L, rgrp, 0)
        _row_chunks(_chunk)

        pltpu.sync_copy(redbuf, slab.at[s])
        plsc.subcore_barrier()

        @pl.when(s == 0)
        def _():
            pltpu.sync_copy(slab, sbuf)
            for t in range(D // L):
                sl = pl.ds(t * L, L)
                tot = zeros16
                for ss in range(NS):
                    tot = tot + (sbuf[ss, sl] - sbuf[ss, pl.ds(D + t * L, L)])
                val_buf[sl] = tot
            pltpu.sync_copy(val_buf.at[pl.ds(0, D)], sum_out.at[c])

    def _side(src_emb, dst_emb, rows2, cols2, vals2, out_hbm, w_ref, sum_out):
        def gbody(g, carry):
            _spmm(g, src_emb, dst_emb, rows2, cols2, vals2, out_hbm)
            plsc.subcore_barrier()
            return carry
        lax.fori_loop(0, NG, gbody, 0)
        _reduce(NG - 1, dst_emb, w_ref, sum_out)
        plsc.subcore_barrier()

    _side(user_emb, item_emb, gu_rows, gu_cols, gu_vals, new_item,
          wcol, user_sums)
    _side(item_emb, user_emb, gi_rows, gi_cols, gi_vals, new_user,
          wrow, item_sums)


def _dense_body(usums, isums, uw, iw, out_u, out_i):
    ue = usums[...] * (1.0 / NU)
    ie = isums[...] * (1.0 / NI)
    mu = jnp.mean(ue, axis=0, keepdims=True)
    mi = jnp.mean(ie, axis=0, keepdims=True)
    pad = jnp.zeros((8 - NB - 1, D), _f32)
    xu = jnp.concatenate([ue, mu, pad], axis=0)
    xi = jnp.concatenate([ie, mi, pad], axis=0)
    out_u[...] = jax.nn.sigmoid(
        jnp.dot(xu, uw[...], preferred_element_type=_f32))
    out_i[...] = jax.nn.sigmoid(
        jnp.dot(xi, iw[...], preferred_element_type=_f32))


def _pad_edges(idx_or_val, pad_col):
    flat = idx_or_val.reshape(NB, NG, E)
    padv = jnp.broadcast_to(pad_col, (NB, NG, EPAD - E))
    return jnp.concatenate([flat, padv], axis=2).reshape(NB, NG, ECH, CH)


@functools.partial(jax.jit)
def kernel(item_embedding_g, user_embedding_g, gi_rows, gi_cols, gi_vals,
           gu_rows, gu_cols, gu_vals, A_rows, A_cols, A_vals, u_w, i_w):
    pad_idx = (jnp.arange(EPAD - E, dtype=_i32) * 37) % NN
    pad_val = jnp.zeros((EPAD - E,), _f32)
    gi_rows2 = _pad_edges(gi_rows.astype(_i32), pad_idx)
    gi_cols2 = _pad_edges(gi_cols.astype(_i32), pad_idx)
    gi_vals2 = _pad_edges(gi_vals, pad_val)
    gu_rows2 = _pad_edges(gu_rows.astype(_i32), pad_idx)
    gu_cols2 = _pad_edges(gu_cols.astype(_i32), pad_idx)
    gu_vals2 = _pad_edges(gu_vals, pad_val)
    apad_idx = (jnp.arange(APAD - E, dtype=_i32) * 53) % NN
    apad_val = jnp.zeros((APAD - E,), _f32)

    def _pad_a(x, padv):
        padv = jnp.broadcast_to(padv, (NB, APAD - E))
        return jnp.concatenate([x, padv], axis=1).reshape(NB, AECH, 64)

    A_rows2 = _pad_a(A_rows.astype(_i32), apad_idx)
    A_cols2 = _pad_a(A_cols.astype(_i32), apad_idx)
    A_vals2 = _pad_a(A_vals, apad_val)

    mesh = plsc.VectorSubcoreMesh(core_axis_name="c", subcore_axis_name="s")
    sc_call = pl.kernel(
        _sc_body,
        out_type=[
            jax.ShapeDtypeStruct((NB, NG, NI, D), _f32),
            jax.ShapeDtypeStruct((NB, NG, NU, D), _f32),
            jax.ShapeDtypeStruct((NB, D), _f32),
            jax.ShapeDtypeStruct((NB, D), _f32),
        ],
        mesh=mesh,
        scratch_types=[
            pltpu.VMEM((WCH,), _f32),
            pltpu.VMEM((2, CPB, CH), _i32),
            pltpu.VMEM((2, CPB, CH), _i32),
            pltpu.VMEM((2, CPB, CH), _f32),
            pltpu.VMEM((CH, D), _f32),
            pltpu.VMEM((CH, D), _f32),
            pltpu.VMEM((CH, D), _f32),
            pltpu.VMEM((CH, D), _f32),
            pltpu.VMEM((WCH,), _f32),
            pltpu.VMEM((2 * D,), _f32),
            pltpu.VMEM((NS, 2 * D), _f32),
            pltpu.VMEM_SHARED((NN, D), _f32),
            pltpu.VMEM_SHARED((NPAD,), _f32),
            pltpu.VMEM_SHARED((NPAD,), _f32),
            pltpu.VMEM_SHARED((NS, 2 * D), _f32),
            pltpu.SemaphoreType.DMA,
            pltpu.SemaphoreType.DMA,
            pltpu.SemaphoreType.DMA,
            pltpu.SemaphoreType.DMA,
            pltpu.SemaphoreType.DMA,
            pltpu.SemaphoreType.DMA,
        ],
    )
    new_item_g, new_user_g, user_sums, item_sums = sc_call(
        item_embedding_g, user_embedding_g, gi_rows2, gi_cols2, gi_vals2,
        gu_rows2, gu_cols2, gu_vals2, A_rows2, A_cols2, A_vals2)

    out_u, out_i = pl.pallas_call(
        _dense_body,
        out_shape=[jax.ShapeDtypeStruct((8, D), _f32),
                   jax.ShapeDtypeStruct((8, D), _f32)],
    )(user_sums, item_sums, u_w, i_w)

    user_embedding = out_u[NB]
    item_embedding = out_i[NB]
    user_embeddings_o = out_u[:NB]
    item_embeddings_o = out_i[:NB]
    return (user_embedding, item_embedding, user_embeddings_o,
            item_embeddings_o, new_item_g, new_user_g)


# speedup vs baseline: 12.7530x; 1.0084x over previous
import functools

import jax
import jax.numpy as jnp
from jax import lax
from jax.experimental import pallas as pl
from jax.experimental.pallas import tpu as pltpu
from jax.experimental.pallas import tpu_sc as plsc

NB = 2
NG = 2
NU = 10000
NI = 10000
NN = 10000
D = 128
E = 320000

NS = 16
L = 16

WCH = 128
WPB = 8
WBPT = 40
AECH = WPB * WBPT * NS
APAD = AECH * 64

CH = 64
CPB = 8
BPT = 40
CPT = CPB * BPT
ECH = CPT * NS
EPAD = ECH * CH

RFULL = NN // CH
RTAIL = NN - RFULL * CH
RITERS = (RFULL + NS - 1) // NS
NPAD = 10240

_f32 = jnp.float32
_i32 = jnp.int32


def _sc_body(item_emb, user_emb, gi_rows, gi_cols, gi_vals,
             gu_rows, gu_cols, gu_vals, a_rows, a_cols, a_vals,
             new_item, new_user, user_sums, item_sums,
             val_buf, cst3, rst3, vst3,
             buf0, buf1, buf2, buf3,
             wbuf, redbuf, sbuf,
             acc, wrow, wcol, slab,
             stsem, sem0, sem1, sem2, sem3, wsem):
    c = lax.axis_index("c")
    s = lax.axis_index("s")
    zeros16 = jnp.zeros((L,), _f32)
    bufs = (buf0, buf1, buf2, buf3)
    sems = (sem0, sem1, sem2, sem3)
    gbuf0 = buf0
    ebuf = buf1

    for t in range(WCH // L):
        wbuf[pl.ds(t * L, L)] = zeros16

    def _zw(k, carry):
        w0 = (s + k * NS) * WCH
        pltpu.sync_copy(wbuf, wrow.at[pl.ds(w0, WCH)])
        pltpu.sync_copy(wbuf, wcol.at[pl.ds(w0, WCH)])
        return carry
    lax.fori_loop(0, NPAD // WCH // NS, _zw, 0)
    plsc.subcore_barrier()

    wstart = s * (WBPT * WPB)

    def _wblk(kb, carry):
        b0 = pl.multiple_of(wstart + kb * WPB, 8)
        d1 = pltpu.async_copy(a_rows.at[c, pl.ds(b0, WPB)], cst3.at[0], stsem)
        d2 = pltpu.async_copy(a_cols.at[c, pl.ds(b0, WPB)], rst3.at[0], stsem)
        d3 = pltpu.async_copy(a_vals.at[c, pl.ds(b0, WPB)], vst3.at[0], stsem)
        d1.wait(); d2.wait(); d3.wait()
        for j in range(WPB):
            pltpu.async_copy(vst3.at[0, j], wrow.at[cst3.at[0, j]], wsem,
                             add=True)
            pltpu.async_copy(vst3.at[0, j], wcol.at[rst3.at[0, j]], wsem,
                             add=True)
        for j in range(WPB):
            pltpu.make_async_copy(vst3.at[0, j], wrow.at[cst3.at[0, j]],
                                  wsem).wait()
            pltpu.make_async_copy(vst3.at[0, j], wcol.at[rst3.at[0, j]],
                                  wsem).wait()
        return carry
    lax.fori_loop(0, WBPT, _wblk, 0)
    plsc.subcore_barrier()

    def _row_chunks(fn):
        def body(k, carry):
            m = s + k * NS
            @pl.when(m < RFULL)
            def _():
                fn(m * CH, CH)
            return carry
        lax.fori_loop(0, RITERS, body, 0)
        @pl.when(s == RFULL % NS)
        def _():
            fn(RFULL * CH, RTAIL)

    def _spmm(g, src_emb, dst_emb, rows2, cols2, vals2, out_hbm):
        def _ibody(k, carry):
            m = s + k * NS
            @pl.when(m < RFULL)
            def _():
                r0 = m * CH
                pltpu.async_copy(dst_emb.at[c, g, pl.ds(r0, CH)],
                                 acc.at[pl.ds(r0, CH)], stsem)
            return carry
        lax.fori_loop(0, RITERS, _ibody, 0)

        def _idrain(k, carry):
            m = s + k * NS
            @pl.when(m < RFULL)
            def _():
                pltpu.make_async_copy(dst_emb.at[c, g, pl.ds(0, CH)],
                                      acc.at[pl.ds(0, CH)], stsem).wait()
            return carry
        lax.fori_loop(0, RITERS, _idrain, 0)
        @pl.when(s == RFULL % NS)
        def _():
            pltpu.sync_copy(dst_emb.at[c, g, pl.ds(RFULL * CH, RTAIL)],
                            acc.at[pl.ds(RFULL * CH, RTAIL)])
        plsc.subcore_barrier()

        start = s * CPT

        def _stage(kbx, par):
            b0 = pl.multiple_of(start + kbx * CPB, 8)
            pltpu.async_copy(cols2.at[c, g, pl.ds(b0, CPB)], cst3.at[par],
                             stsem)
            pltpu.async_copy(rows2.at[c, g, pl.ds(b0, CPB)], rst3.at[par],
                             stsem)
            pltpu.async_copy(vals2.at[c, g, pl.ds(b0, CPB)], vst3.at[par],
                             stsem)

        def _stage_wait(par):
            for ref, arr in ((cst3, cols2), (rst3, rows2), (vst3, vals2)):
                pltpu.make_async_copy(arr.at[c, g, pl.ds(0, CPB)],
                                      ref.at[par], stsem).wait()

        def _gissue(par, row, slot):
            pltpu.async_copy(src_emb.at[c, g].at[cst3.at[par, row]],
                             bufs[slot], sems[slot])

        def _gwait(par, row, slot):
            pltpu.make_async_copy(src_emb.at[c, g].at[cst3.at[par, row]],
                                  bufs[slot], sems[slot]).wait()

        def _sissue(par, row, slot):
            pltpu.async_copy(bufs[slot], acc.at[rst3.at[par, row]],
                             sems[slot], add=True)

        def _sdrain(par, row, slot):
            pltpu.make_async_copy(bufs[slot], acc.at[rst3.at[par, row]],
                                  sems[slot]).wait()

        _stage(0, 0)
        _stage_wait(0)
        _gissue(0, 0, 0)
        _gissue(0, 1, 1)

        def blk(kb, carry):
            hh = lax.rem(kb, 2)

            def quad(q, inner):
                for i in range(4):
                    j = q * 4 + i
                    slot = i
                    nslot = (i + 2) % 4
                    if i < 2:
                        @pl.when((kb > 0) | (q > 0))
                        def _():
                            _sdrain(hh, j, nslot)
                    else:
                        _sdrain(hh, j, nslot)
                    if i < 2:
                        _gissue(hh, j + 2, nslot)
                        if i == 1:
                            @pl.when((q == 0) & (kb < BPT - 1))
                            def _():
                                _stage(kb + 1, 1 - hh)
                    else:
                        @pl.when(q == 0)
                        def _():
                            _gissue(hh, j + 2, nslot)
                        if i == 2:
                            @pl.when((q == 1) & (kb < BPT - 1))
                            def _():
                                _stage_wait(1 - hh)
                                _gissue(1 - hh, 0, 0)
                        else:
                            @pl.when((q == 1) & (kb < BPT - 1))
                            def _():
                                _gissue(1 - hh, 1, 1)
                    _gwait(hh, j, slot)
                    gb = bufs[slot]

                    def sgrp(jj, sinner):
                        vv = vst3[hh, j, pl.ds(jj * L, L)]
                        for l in range(L):
                            v = vv[l]
                            row = jj * L + l
                            for t in range(D // L):
                                sl = pl.ds(t * L, L)
                                gb[row, sl] = gb[row, sl] * v
                        return sinner
                    lax.fori_loop(0, CH // L, sgrp, 0)
                    _sissue(hh, j, slot)
                return inner
            lax.fori_loop(0, CPB // 4, quad, 0)
            return carry
        lax.fori_loop(0, BPT, blk, 0)
        hlast = (BPT - 1) % 2
        _sdrain(hlast, 6, 2)
        _sdrain(hlast, 7, 3)
        plsc.subcore_barrier()

        def _wbody(k, carry):
            m = s + k * NS
            @pl.when(m < RFULL)
            def _():
                r0 = m * CH
                pltpu.async_copy(acc.at[pl.ds(r0, CH)],
                                 out_hbm.at[c, g, pl.ds(r0, CH)], stsem)
            return carry
        lax.fori_loop(0, RITERS, _wbody, 0)

        def _wdrain2(k, carry):
            m = s + k * NS
            @pl.when(m < RFULL)
            def _():
                pltpu.make_async_copy(acc.at[pl.ds(0, CH)],
                                      out_hbm.at[c, g, pl.ds(0, CH)],
                                      stsem).wait()
            return carry
        lax.fori_loop(0, RITERS, _wdrain2, 0)
        @pl.when(s == RFULL % NS)
        def _():
            pltpu.sync_copy(acc.at[pl.ds(RFULL * CH, RTAIL)],
                            out_hbm.at[c, g, pl.ds(RFULL * CH, RTAIL)])

    def _reduce(g, dst_emb, w_ref, sum_out):
        for t in range(2 * D // L):
            redbuf[pl.ds(t * L, L)] = zeros16

        def _chunk(r0, n):
            pltpu.sync_copy(acc.at[pl.ds(r0, n)], gbuf0.at[pl.ds(0, n)])
            pltpu.sync_copy(dst_emb.at[c, g, pl.ds(r0, n)], ebuf.at[pl.ds(0, n)])
            pltpu.sync_copy(w_ref.at[pl.ds(r0, n)], wbuf.at[pl.ds(0, n)])

            def rgrp(jj, carry):
                wv = wbuf[pl.ds(jj * L, L)]
                for l in range(L):
                    w = wv[l]
                    row = jj * L + l
                    for t in range(D // L):
                        sl = pl.ds(t * L, L)
                        plsc.addupdate(redbuf.at[sl], gbuf0[row, sl] * w)
                        plsc.addupdate(redbuf.at[pl.ds(D + t * L, L)],
                                       ebuf[row, sl] * w)
                return carry
            lax.fori_loop(0, n // L, rgrp, 0)
        _row_chunks(_chunk)

        pltpu.sync_copy(redbuf, slab.at[s])
        plsc.subcore_barrier()

        @pl.when(s == 0)
        def _():
            pltpu.sync_copy(slab, sbuf)
            for t in range(D // L):
                sl = pl.ds(t * L, L)
                tot = zeros16
                for ss in range(NS):
                    tot = tot + (sbuf[ss, sl] - sbuf[ss, pl.ds(D + t * L, L)])
                val_buf[sl] = tot
            pltpu.sync_copy(val_buf.at[pl.ds(0, D)], sum_out.at[c])

    def _side(src_emb, dst_emb, rows2, cols2, vals2, out_hbm, w_ref, sum_out):
        def gbody(g, carry):
            _spmm(g, src_emb, dst_emb, rows2, cols2, vals2, out_hbm)
            plsc.subcore_barrier()
            return carry
        lax.fori_loop(0, NG, gbody, 0)
        _reduce(NG - 1, dst_emb, w_ref, sum_out)
        plsc.subcore_barrier()

    _side(user_emb, item_emb, gu_rows, gu_cols, gu_vals, new_item,
          wcol, user_sums)
    _side(item_emb, user_emb, gi_rows, gi_cols, gi_vals, new_user,
          wrow, item_sums)


def _dense_body(usums, isums, uw, iw, out_u, out_i):
    ue = usums[...] * (1.0 / NU)
    ie = isums[...] * (1.0 / NI)
    mu = jnp.mean(ue, axis=0, keepdims=True)
    mi = jnp.mean(ie, axis=0, keepdims=True)
    pad = jnp.zeros((8 - NB - 1, D), _f32)
    xu = jnp.concatenate([ue, mu, pad], axis=0)
    xi = jnp.concatenate([ie, mi, pad], axis=0)
    out_u[...] = jax.nn.sigmoid(
        jnp.dot(xu, uw[...], preferred_element_type=_f32))
    out_i[...] = jax.nn.sigmoid(
        jnp.dot(xi, iw[...], preferred_element_type=_f32))


def _pad_edges(idx_or_val, pad_col):
    flat = idx_or_val.reshape(NB, NG, E)
    padv = jnp.broadcast_to(pad_col, (NB, NG, EPAD - E))
    return jnp.concatenate([flat, padv], axis=2).reshape(NB, NG, ECH, CH)


@functools.partial(jax.jit)
def kernel(item_embedding_g, user_embedding_g, gi_rows, gi_cols, gi_vals,
           gu_rows, gu_cols, gu_vals, A_rows, A_cols, A_vals, u_w, i_w):
    pad_idx = (jnp.arange(EPAD - E, dtype=_i32) * 37) % NN
    pad_val = jnp.zeros((EPAD - E,), _f32)
    gi_rows2 = _pad_edges(gi_rows.astype(_i32), pad_idx)
    gi_cols2 = _pad_edges(gi_cols.astype(_i32), pad_idx)
    gi_vals2 = _pad_edges(gi_vals, pad_val)
    gu_rows2 = _pad_edges(gu_rows.astype(_i32), pad_idx)
    gu_cols2 = _pad_edges(gu_cols.astype(_i32), pad_idx)
    gu_vals2 = _pad_edges(gu_vals, pad_val)
    apad_idx = (jnp.arange(APAD - E, dtype=_i32) * 53) % NN
    apad_val = jnp.zeros((APAD - E,), _f32)

    def _pad_a(x, padv):
        padv = jnp.broadcast_to(padv, (NB, APAD - E))
        return jnp.concatenate([x, padv], axis=1).reshape(NB, AECH, 64)

    A_rows2 = _pad_a(A_rows.astype(_i32), apad_idx)
    A_cols2 = _pad_a(A_cols.astype(_i32), apad_idx)
    A_vals2 = _pad_a(A_vals, apad_val)

    mesh = plsc.VectorSubcoreMesh(core_axis_name="c", subcore_axis_name="s")
    sc_call = pl.kernel(
        _sc_body,
        out_type=[
            jax.ShapeDtypeStruct((NB, NG, NI, D), _f32),
            jax.ShapeDtypeStruct((NB, NG, NU, D), _f32),
            jax.ShapeDtypeStruct((NB, D), _f32),
            jax.ShapeDtypeStruct((NB, D), _f32),
        ],
        mesh=mesh,
        scratch_types=[
            pltpu.VMEM((WCH,), _f32),
            pltpu.VMEM((2, CPB, CH), _i32),
            pltpu.VMEM((2, CPB, CH), _i32),
            pltpu.VMEM((2, CPB, CH), _f32),
            pltpu.VMEM((CH, D), _f32),
            pltpu.VMEM((CH, D), _f32),
            pltpu.VMEM((CH, D), _f32),
            pltpu.VMEM((CH, D), _f32),
            pltpu.VMEM((WCH,), _f32),
            pltpu.VMEM((2 * D,), _f32),
            pltpu.VMEM((NS, 2 * D), _f32),
            pltpu.VMEM_SHARED((NN, D), _f32),
            pltpu.VMEM_SHARED((NPAD,), _f32),
            pltpu.VMEM_SHARED((NPAD,), _f32),
            pltpu.VMEM_SHARED((NS, 2 * D), _f32),
            pltpu.SemaphoreType.DMA,
            pltpu.SemaphoreType.DMA,
            pltpu.SemaphoreType.DMA,
            pltpu.SemaphoreType.DMA,
            pltpu.SemaphoreType.DMA,
            pltpu.SemaphoreType.DMA,
        ],
    )
    new_item_g, new_user_g, user_sums, item_sums = sc_call(
        item_embedding_g, user_embedding_g, gi_rows2, gi_cols2, gi_vals2,
        gu_rows2, gu_cols2, gu_vals2, A_rows2, A_cols2, A_vals2)

    out_u, out_i = pl.pallas_call(
        _dense_body,
        out_shape=[jax.ShapeDtypeStruct((8, D), _f32),
                   jax.ShapeDtypeStruct((8, D), _f32)],
    )(user_sums, item_sums, u_w, i_w)

    user_embedding = out_u[NB]
    item_embedding = out_i[NB]
    user_embeddings_o = out_u[:NB]
    item_embeddings_o = out_i[:NB]
    return (user_embedding, item_embedding, user_embeddings_o,
            item_embeddings_o, new_item_g, new_user_g)


# speedup vs baseline: 12.8025x; 1.0039x over previous
import functools

import jax
import jax.numpy as jnp
from jax import lax
from jax.experimental import pallas as pl
from jax.experimental.pallas import tpu as pltpu
from jax.experimental.pallas import tpu_sc as plsc

NB = 2
NG = 2
NU = 10000
NI = 10000
NN = 10000
D = 128
E = 320000

NS = 16
L = 16

WCH = 128
WPB = 8
WBPT = 40
AECH = WPB * WBPT * NS
APAD = AECH * 64

CH = 64
CPB = 8
BPT = 40
CPT = CPB * BPT
ECH = CPT * NS
EPAD = ECH * CH

RFULL = NN // CH
RTAIL = NN - RFULL * CH
RITERS = (RFULL + NS - 1) // NS
NPAD = 10240

_f32 = jnp.float32
_i32 = jnp.int32


def _sc_body(item_emb, user_emb, gi_rows, gi_cols, gi_vals,
             gu_rows, gu_cols, gu_vals, a_rows, a_cols, a_vals,
             new_item, new_user, user_sums, item_sums,
             val_buf, cst3, rst3, vst3,
             buf0, buf1, buf2, buf3,
             wbuf, redbuf, sbuf,
             acc, wrow, wcol, slab,
             stsem, sem0, sem1, sem2, sem3, wsem):
    c = lax.axis_index("c")
    s = lax.axis_index("s")
    zeros16 = jnp.zeros((L,), _f32)
    bufs = (buf0, buf1, buf2, buf3)
    sems = (sem0, sem1, sem2, sem3)
    gbuf0 = buf0
    ebuf = buf1

    for t in range(WCH // L):
        wbuf[pl.ds(t * L, L)] = zeros16

    def _zw(k, carry):
        w0 = (s + k * NS) * WCH
        pltpu.sync_copy(wbuf, wrow.at[pl.ds(w0, WCH)])
        pltpu.sync_copy(wbuf, wcol.at[pl.ds(w0, WCH)])
        return carry
    lax.fori_loop(0, NPAD // WCH // NS, _zw, 0)
    plsc.subcore_barrier()

    wstart = s * (WBPT * WPB)

    def _wblk(kb, carry):
        b0 = pl.multiple_of(wstart + kb * WPB, 8)
        d1 = pltpu.async_copy(a_rows.at[c, pl.ds(b0, WPB)], cst3.at[0], stsem)
        d2 = pltpu.async_copy(a_cols.at[c, pl.ds(b0, WPB)], rst3.at[0], stsem)
        d3 = pltpu.async_copy(a_vals.at[c, pl.ds(b0, WPB)], vst3.at[0], stsem)
        d1.wait(); d2.wait(); d3.wait()
        for j in range(WPB):
            pltpu.async_copy(vst3.at[0, j], wrow.at[cst3.at[0, j]], wsem,
                             add=True)
            pltpu.async_copy(vst3.at[0, j], wcol.at[rst3.at[0, j]], wsem,
                             add=True)
        for j in range(WPB):
            pltpu.make_async_copy(vst3.at[0, j], wrow.at[cst3.at[0, j]],
                                  wsem).wait()
            pltpu.make_async_copy(vst3.at[0, j], wcol.at[rst3.at[0, j]],
                                  wsem).wait()
        return carry
    lax.fori_loop(0, WBPT, _wblk, 0)
    plsc.subcore_barrier()

    def _row_chunks(fn):
        def body(k, carry):
            m = s + k * NS
            @pl.when(m < RFULL)
            def _():
                fn(m * CH, CH)
            return carry
        lax.fori_loop(0, RITERS, body, 0)
        @pl.when(s == RFULL % NS)
        def _():
            fn(RFULL * CH, RTAIL)

    def _spmm(g, src_emb, dst_emb, rows2, cols2, vals2, out_hbm):
        def _ibody(k, carry):
            m = s + k * NS
            @pl.when(m < RFULL)
            def _():
                r0 = m * CH
                pltpu.async_copy(dst_emb.at[c, g, pl.ds(r0, CH)],
                                 acc.at[pl.ds(r0, CH)], stsem)
            return carry
        lax.fori_loop(0, RITERS, _ibody, 0)

        def _idrain(k, carry):
            m = s + k * NS
            @pl.when(m < RFULL)
            def _():
                pltpu.make_async_copy(dst_emb.at[c, g, pl.ds(0, CH)],
                                      acc.at[pl.ds(0, CH)], stsem).wait()
            return carry
        lax.fori_loop(0, RITERS, _idrain, 0)
        @pl.when(s == RFULL % NS)
        def _():
            pltpu.sync_copy(dst_emb.at[c, g, pl.ds(RFULL * CH, RTAIL)],
                            acc.at[pl.ds(RFULL * CH, RTAIL)])
        plsc.subcore_barrier()

        start = s * CPT

        def _stage(kbx, par):
            b0 = pl.multiple_of(start + kbx * CPB, 8)
            pltpu.async_copy(cols2.at[c, g, pl.ds(b0, CPB)], cst3.at[par],
                             stsem)
            pltpu.async_copy(rows2.at[c, g, pl.ds(b0, CPB)], rst3.at[par],
                             stsem)
            pltpu.async_copy(vals2.at[c, g, pl.ds(b0, CPB)], vst3.at[par],
                             stsem)

        def _stage_wait(par):
            for ref, arr in ((cst3, cols2), (rst3, rows2), (vst3, vals2)):
                pltpu.make_async_copy(arr.at[c, g, pl.ds(0, CPB)],
                                      ref.at[par], stsem).wait()

        def _gissue(par, row, slot):
            pltpu.async_copy(src_emb.at[c, g].at[cst3.at[par, row]],
                             bufs[slot], sems[slot])

        def _gwait(par, row, slot):
            pltpu.make_async_copy(src_emb.at[c, g].at[cst3.at[par, row]],
                                  bufs[slot], sems[slot]).wait()

        def _sissue(par, row, slot):
            pltpu.async_copy(bufs[slot], acc.at[rst3.at[par, row]],
                             sems[slot], add=True)

        def _sdrain(par, row, slot):
            pltpu.make_async_copy(bufs[slot], acc.at[rst3.at[par, row]],
                                  sems[slot]).wait()

        _stage(0, 0)
        _stage_wait(0)
        _gissue(0, 0, 0)
        _gissue(0, 1, 1)

        def blk(kb, carry):
            hh = lax.rem(kb, 2)

            def quad(q, inner):
                for i in range(4):
                    j = q * 4 + i
                    slot = i
                    nslot = (i + 2) % 4
                    if i < 2:
                        @pl.when((kb > 0) | (q > 0))
                        def _():
                            _sdrain(hh, j, nslot)
                    else:
                        _sdrain(hh, j, nslot)
                    if i < 2:
                        _gissue(hh, j + 2, nslot)
                        if i == 1:
                            @pl.when((q == 0) & (kb < BPT - 1))
                            def _():
                                _stage(kb + 1, 1 - hh)
                    else:
                        @pl.when(q == 0)
                        def _():
                            _gissue(hh, j + 2, nslot)
                        if i == 2:
                            @pl.when((q == 1) & (kb < BPT - 1))
                            def _():
                                _stage_wait(1 - hh)
                                _gissue(1 - hh, 0, 0)
                        else:
                            @pl.when((q == 1) & (kb < BPT - 1))
                            def _():
                                _gissue(1 - hh, 1, 1)
                    _gwait(hh, j, slot)
                    gb = bufs[slot]

                    @plsc.parallel_loop(0, CH // L, unroll=2)
                    def _sg(jj):
                        vv = vst3[hh, j, pl.ds(jj * L, L)]
                        for l in range(L):
                            v = vv[l]
                            row = jj * L + l
                            for t in range(D // L):
                                sl = pl.ds(t * L, L)
                                gb[row, sl] = gb[row, sl] * v
                    _sissue(hh, j, slot)
                return inner
            lax.fori_loop(0, CPB // 4, quad, 0)
            return carry
        lax.fori_loop(0, BPT, blk, 0)
        hlast = (BPT - 1) % 2
        _sdrain(hlast, 6, 2)
        _sdrain(hlast, 7, 3)
        plsc.subcore_barrier()

        def _wbody(k, carry):
            m = s + k * NS
            @pl.when(m < RFULL)
            def _():
                r0 = m * CH
                pltpu.async_copy(acc.at[pl.ds(r0, CH)],
                                 out_hbm.at[c, g, pl.ds(r0, CH)], stsem)
            return carry
        lax.fori_loop(0, RITERS, _wbody, 0)

        def _wdrain2(k, carry):
            m = s + k * NS
            @pl.when(m < RFULL)
            def _():
                pltpu.make_async_copy(acc.at[pl.ds(0, CH)],
                                      out_hbm.at[c, g, pl.ds(0, CH)],
                                      stsem).wait()
            return carry
        lax.fori_loop(0, RITERS, _wdrain2, 0)
        @pl.when(s == RFULL % NS)
        def _():
            pltpu.sync_copy(acc.at[pl.ds(RFULL * CH, RTAIL)],
                            out_hbm.at[c, g, pl.ds(RFULL * CH, RTAIL)])

    def _reduce(g, dst_emb, w_ref, sum_out):
        for t in range(2 * D // L):
            redbuf[pl.ds(t * L, L)] = zeros16

        def _chunk(r0, n):
            pltpu.sync_copy(acc.at[pl.ds(r0, n)], gbuf0.at[pl.ds(0, n)])
            pltpu.sync_copy(dst_emb.at[c, g, pl.ds(r0, n)], ebuf.at[pl.ds(0, n)])
            pltpu.sync_copy(w_ref.at[pl.ds(r0, n)], wbuf.at[pl.ds(0, n)])

            def rgrp(jj, carry):
                wv = wbuf[pl.ds(jj * L, L)]
                for l in range(L):
                    w = wv[l]
                    row = jj * L + l
                    for t in range(D // L):
                        sl = pl.ds(t * L, L)
                        plsc.addupdate(redbuf.at[sl], gbuf0[row, sl] * w)
                        plsc.addupdate(redbuf.at[pl.ds(D + t * L, L)],
                                       ebuf[row, sl] * w)
                return carry
            lax.fori_loop(0, n // L, rgrp, 0)
        _row_chunks(_chunk)

        pltpu.sync_copy(redbuf, slab.at[s])
        plsc.subcore_barrier()

        @pl.when(s == 0)
        def _():
            pltpu.sync_copy(slab, sbuf)
            for t in range(D // L):
                sl = pl.ds(t * L, L)
                tot = zeros16
                for ss in range(NS):
                    tot = tot + (sbuf[ss, sl] - sbuf[ss, pl.ds(D + t * L, L)])
                val_buf[sl] = tot
            pltpu.sync_copy(val_buf.at[pl.ds(0, D)], sum_out.at[c])

    def _side(src_emb, dst_emb, rows2, cols2, vals2, out_hbm, w_ref, sum_out):
        def gbody(g, carry):
            _spmm(g, src_emb, dst_emb, rows2, cols2, vals2, out_hbm)
            plsc.subcore_barrier()
            return carry
        lax.fori_loop(0, NG, gbody, 0)
        _reduce(NG - 1, dst_emb, w_ref, sum_out)
        plsc.subcore_barrier()

    _side(user_emb, item_emb, gu_rows, gu_cols, gu_vals, new_item,
          wcol, user_sums)
    _side(item_emb, user_emb, gi_rows, gi_cols, gi_vals, new_user,
          wrow, item_sums)


def _dense_body(usums, isums, uw, iw, out_u, out_i):
    ue = usums[...] * (1.0 / NU)
    ie = isums[...] * (1.0 / NI)
    mu = jnp.mean(ue, axis=0, keepdims=True)
    mi = jnp.mean(ie, axis=0, keepdims=True)
    pad = jnp.zeros((8 - NB - 1, D), _f32)
    xu = jnp.concatenate([ue, mu, pad], axis=0)
    xi = jnp.concatenate([ie, mi, pad], axis=0)
    out_u[...] = jax.nn.sigmoid(
        jnp.dot(xu, uw[...], preferred_element_type=_f32))
    out_i[...] = jax.nn.sigmoid(
        jnp.dot(xi, iw[...], preferred_element_type=_f32))


def _pad_edges(idx_or_val, pad_col):
    flat = idx_or_val.reshape(NB, NG, E)
    padv = jnp.broadcast_to(pad_col, (NB, NG, EPAD - E))
    return jnp.concatenate([flat, padv], axis=2).reshape(NB, NG, ECH, CH)


@functools.partial(jax.jit)
def kernel(item_embedding_g, user_embedding_g, gi_rows, gi_cols, gi_vals,
           gu_rows, gu_cols, gu_vals, A_rows, A_cols, A_vals, u_w, i_w):
    pad_idx = (jnp.arange(EPAD - E, dtype=_i32) * 37) % NN
    pad_val = jnp.zeros((EPAD - E,), _f32)
    gi_rows2 = _pad_edges(gi_rows.astype(_i32), pad_idx)
    gi_cols2 = _pad_edges(gi_cols.astype(_i32), pad_idx)
    gi_vals2 = _pad_edges(gi_vals, pad_val)
    gu_rows2 = _pad_edges(gu_rows.astype(_i32), pad_idx)
    gu_cols2 = _pad_edges(gu_cols.astype(_i32), pad_idx)
    gu_vals2 = _pad_edges(gu_vals, pad_val)
    apad_idx = (jnp.arange(APAD - E, dtype=_i32) * 53) % NN
    apad_val = jnp.zeros((APAD - E,), _f32)

    def _pad_a(x, padv):
        padv = jnp.broadcast_to(padv, (NB, APAD - E))
        return jnp.concatenate([x, padv], axis=1).reshape(NB, AECH, 64)

    A_rows2 = _pad_a(A_rows.astype(_i32), apad_idx)
    A_cols2 = _pad_a(A_cols.astype(_i32), apad_idx)
    A_vals2 = _pad_a(A_vals, apad_val)

    mesh = plsc.VectorSubcoreMesh(core_axis_name="c", subcore_axis_name="s")
    sc_call = pl.kernel(
        _sc_body,
        out_type=[
            jax.ShapeDtypeStruct((NB, NG, NI, D), _f32),
            jax.ShapeDtypeStruct((NB, NG, NU, D), _f32),
            jax.ShapeDtypeStruct((NB, D), _f32),
            jax.ShapeDtypeStruct((NB, D), _f32),
        ],
        mesh=mesh,
        scratch_types=[
            pltpu.VMEM((WCH,), _f32),
            pltpu.VMEM((2, CPB, CH), _i32),
            pltpu.VMEM((2, CPB, CH), _i32),
            pltpu.VMEM((2, CPB, CH), _f32),
            pltpu.VMEM((CH, D), _f32),
            pltpu.VMEM((CH, D), _f32),
            pltpu.VMEM((CH, D), _f32),
            pltpu.VMEM((CH, D), _f32),
            pltpu.VMEM((WCH,), _f32),
            pltpu.VMEM((2 * D,), _f32),
            pltpu.VMEM((NS, 2 * D), _f32),
            pltpu.VMEM_SHARED((NN, D), _f32),
            pltpu.VMEM_SHARED((NPAD,), _f32),
            pltpu.VMEM_SHARED((NPAD,), _f32),
            pltpu.VMEM_SHARED((NS, 2 * D), _f32),
            pltpu.SemaphoreType.DMA,
            pltpu.SemaphoreType.DMA,
            pltpu.SemaphoreType.DMA,
            pltpu.SemaphoreType.DMA,
            pltpu.SemaphoreType.DMA,
            pltpu.SemaphoreType.DMA,
        ],
    )
    new_item_g, new_user_g, user_sums, item_sums = sc_call(
        item_embedding_g, user_embedding_g, gi_rows2, gi_cols2, gi_vals2,
        gu_rows2, gu_cols2, gu_vals2, A_rows2, A_cols2, A_vals2)

    out_u, out_i = pl.pallas_call(
        _dense_body,
        out_shape=[jax.ShapeDtypeStruct((8, D), _f32),
                   jax.ShapeDtypeStruct((8, D), _f32)],
    )(user_sums, item_sums, u_w, i_w)

    user_embedding = out_u[NB]
    item_embedding = out_i[NB]
    user_embeddings_o = out_u[:NB]
    item_embeddings_o = out_i[:NB]
    return (user_embedding, item_embedding, user_embeddings_o,
            item_embeddings_o, new_item_g, new_user_g)
